# baseline (speedup 1.0000x reference)
import functools

import jax
import jax.numpy as jnp
from jax import lax
from jax.experimental import pallas as pl
from jax.experimental.pallas import tpu as pltpu
from jax.experimental.pallas import tpu_sc as plsc

_N = 10000
_NP = _N // 2
_DIM = 128
_HD = _DIM // 2
_NC = 2
_NS = 16
_CHUNK = 128
_LANES = 16

_PROWS_BASE = 312
_PROWS_LAST = _NP - 15 * _PROWS_BASE


def _sc_aggregate(xp, row2, col2, vals, pp):
    e_pad = row2.shape[0]
    ept = e_pad // _NS
    n_chunks = ept // _CHUNK

    mesh = plsc.VectorSubcoreMesh(core_axis_name="c", subcore_axis_name="s")

    @functools.partial(
        pl.kernel,
        out_type=jax.ShapeDtypeStruct((_NC, _NP, _DIM), jnp.float32),
        mesh=mesh,
        scratch_types=[
            pltpu.VMEM((_CHUNK,), jnp.int32),
            pltpu.VMEM((_CHUNK,), jnp.int32),
            pltpu.VMEM((_CHUNK,), jnp.float32),
            pltpu.VMEM((_CHUNK,), jnp.int32),
            pltpu.VMEM((_CHUNK, _DIM), jnp.float32),
            pltpu.VMEM_SHARED((_NP, _DIM), jnp.float32),
            pltpu.VMEM_SHARED((_NP, _DIM), jnp.float32),
            pltpu.SemaphoreType.DMA,
        ],
    )
    def agg(xp_hbm, row_hbm, col_hbm, vals_hbm, pp_hbm, out_hbm,
            cidx_v, ridx_v, vals_v, pp_v, rows_v, xp_sh, h_sh, sem):
        cid = lax.axis_index("c")
        sid = lax.axis_index("s")

        base_row = sid * _PROWS_BASE

        def _stage(off, cnt):
            pltpu.sync_copy(xp_hbm.at[cid, pl.ds(base_row + off, cnt)],
                            rows_v.at[pl.ds(0, cnt)])
            pltpu.sync_copy(rows_v.at[pl.ds(0, cnt)],
                            xp_sh.at[pl.ds(base_row + off, cnt)])

        for off in range(0, _PROWS_BASE - _CHUNK + 1, _CHUNK):
            _stage(off, _CHUNK)
        _stage_off = (_PROWS_BASE // _CHUNK) * _CHUNK

        @pl.when(sid < _NS - 1)
        def _stage_tail():
            _stage(_stage_off, _PROWS_BASE - _stage_off)

        @pl.when(sid == _NS - 1)
        def _stage_tail_last():
            _stage(_stage_off, _PROWS_LAST - _stage_off)

        def _zero_row(i, carry):
            for j in range(_DIM // _LANES):
                rows_v[i, pl.ds(j * _LANES, _LANES)] = jnp.zeros(
                    (_LANES,), jnp.float32)
            return carry
        lax.fori_loop(0, _CHUNK, _zero_row, 0)

        for off in range(0, _PROWS_BASE - _CHUNK + 1, _CHUNK):
            pltpu.sync_copy(rows_v, h_sh.at[pl.ds(base_row + off, _CHUNK)])

        @pl.when(sid < _NS - 1)
        def _zero_tail():
            pltpu.sync_copy(rows_v.at[pl.ds(0, _PROWS_BASE - _stage_off)],
                            h_sh.at[pl.ds(base_row + _stage_off,
                                          _PROWS_BASE - _stage_off)])

        @pl.when(sid == _NS - 1)
        def _zero_tail_last():
            pltpu.sync_copy(rows_v.at[pl.ds(0, _PROWS_LAST - _stage_off)],
                            h_sh.at[pl.ds(base_row + _stage_off,
                                          _PROWS_LAST - _stage_off)])
        plsc.subcore_barrier()

        def body(c, carry):
            base = sid * ept + c * _CHUNK
            pltpu.sync_copy(col_hbm.at[pl.ds(base, _CHUNK)], cidx_v)
            pltpu.sync_copy(row_hbm.at[pl.ds(base, _CHUNK)], ridx_v)
            pltpu.sync_copy(vals_hbm.at[pl.ds(base, _CHUNK)], vals_v)
            pltpu.sync_copy(pp_hbm.at[pl.ds(base, _CHUNK)], pp_v)
            pltpu.async_copy(xp_sh.at[cidx_v], rows_v, sem).wait()

            def scale(g, inner):
                val16 = vals_v[pl.ds(g * _LANES, _LANES)]
                p16 = pp_v[pl.ds(g * _LANES, _LANES)]
                for e in range(_LANES):
                    bc = val16[e]
                    pe = p16[e]
                    coff = (pe & 1) * _HD
                    roff = (pe >> 1) * _HD
                    zoff = _HD - roff
                    r = g * _LANES + e
                    vregs = [rows_v[r, pl.ds(coff + j * _LANES, _LANES)] * bc
                             for j in range(_HD // _LANES)]
                    for j in range(_HD // _LANES):
                        rows_v[r, pl.ds(roff + j * _LANES, _LANES)] = vregs[j]
                    for j in range(_HD // _LANES):
                        rows_v[r, pl.ds(zoff + j * _LANES, _LANES)] = (
                            jnp.zeros((_LANES,), jnp.float32))
                return inner
            lax.fori_loop(0, _CHUNK // _LANES, scale, 0)

            pltpu.sync_copy(rows_v, h_sh.at[ridx_v], add=True)
            return carry
        lax.fori_loop(0, n_chunks, body, 0)

        plsc.subcore_barrier()

        @pl.when(sid < _NS - 1)
        def _write_base():
            pltpu.sync_copy(h_sh.at[pl.ds(base_row, _PROWS_BASE)],
                            out_hbm.at[cid, pl.ds(base_row, _PROWS_BASE)])

        @pl.when(sid == _NS - 1)
        def _write_last():
            pltpu.sync_copy(h_sh.at[pl.ds(base_row, _PROWS_LAST)],
                            out_hbm.at[cid, pl.ds(base_row, _PROWS_LAST)])

    return agg(xp, row2, col2, vals, pp)


_BN = 1000


def _gate_body(hp_ref, x_ref, w_ref, b_ref, o_ref):
    h = jnp.concatenate([hp_ref[0], hp_ref[1]], axis=1)
    z = jnp.sum(h * w_ref[...], axis=1, keepdims=True) + b_ref[0, 0]
    g = jax.nn.sigmoid(z)
    o_ref[...] = g * h + (1.0 - g) * x_ref[...]


def _gate(hp, x, W_gate, b_gate):
    wt = W_gate.reshape(1, _DIM)
    bb = b_gate.reshape(1, 1)
    grid = _N // _BN
    return pl.pallas_call(
        _gate_body,
        grid=(grid,),
        in_specs=[
            pl.BlockSpec((_NC, _BN, _HD), lambda i: (0, i, 0)),
            pl.BlockSpec((_BN, _DIM), lambda i: (i, 0)),
            pl.BlockSpec((1, _DIM), lambda i: (0, 0)),
            pl.BlockSpec(memory_space=pltpu.SMEM),
        ],
        out_specs=pl.BlockSpec((_BN, _DIM), lambda i: (i, 0)),
        out_shape=jax.ShapeDtypeStruct((_N, _DIM), jnp.float32),
    )(hp, x, wt, bb)


def kernel(x, adj_indices, adj_values, W_gate, b_gate):
    row = adj_indices[0].astype(jnp.int32)
    col = adj_indices[1].astype(jnp.int32)
    vals = adj_values.astype(jnp.float32)
    e = row.shape[0]
    unit = _NS * _CHUNK
    e_pad = ((e + unit - 1) // unit) * unit
    pad = e_pad - e
    if pad:
        spread = jnp.arange(pad, dtype=jnp.int32) % _N
        row = jnp.concatenate([row, spread])
        col = jnp.concatenate([col, spread])
        vals = jnp.concatenate([vals, jnp.zeros((pad,), jnp.float32)])
    row2 = row >> 1
    col2 = col >> 1
    pp = (row & 1) * 2 + (col & 1)
    xp = x.reshape(_NP, 2, _NC, _HD).transpose(2, 0, 1, 3).reshape(
        _NC, _NP, _DIM)
    hp3 = _sc_aggregate(xp, row2, col2, vals, pp)
    hp = hp3.reshape(_NC, _N, _HD)
    return _gate(hp, x, W_gate, b_gate)


# speedup vs baseline: 2.2121x; 2.2121x over previous
import functools

import jax
import jax.numpy as jnp
from jax import lax
from jax.experimental import pallas as pl
from jax.experimental.pallas import tpu as pltpu
from jax.experimental.pallas import tpu_sc as plsc

_N = 10000
_DIM = 128
_NC = 2
_NS = 16
_NW = _NC * _NS
_CHUNK = 128
_LANES = 16

_ROWS_BASE = 624
_ROWS_LAST = _N - 15 * _ROWS_BASE


def _sc_aggregate(x, row, col, vals):
    e_pad = row.shape[0]
    ept = e_pad // _NW
    n_chunks = ept // _CHUNK

    mesh = plsc.VectorSubcoreMesh(core_axis_name="c", subcore_axis_name="s")

    @functools.partial(
        pl.kernel,
        out_type=jax.ShapeDtypeStruct((_NC, _N, _DIM), jnp.float32),
        mesh=mesh,
        scratch_types=[
            pltpu.VMEM((_CHUNK,), jnp.int32),
            pltpu.VMEM((_CHUNK,), jnp.int32),
            pltpu.VMEM((_CHUNK,), jnp.float32),
            pltpu.VMEM((_CHUNK, _DIM), jnp.float32),
            pltpu.VMEM_SHARED((_N, _DIM), jnp.float32),
            pltpu.SemaphoreType.DMA,
        ],
    )
    def agg(x_hbm, row_hbm, col_hbm, vals_hbm, out_hbm,
            cidx_v, ridx_v, vals_v, rows_v, h_sh, sem):
        cid = lax.axis_index("c")
        sid = lax.axis_index("s")
        wid = sid * _NC + cid

        def _zero_row(i, carry):
            for j in range(_DIM // _LANES):
                rows_v[i, pl.ds(j * _LANES, _LANES)] = jnp.zeros(
                    (_LANES,), jnp.float32)
            return carry
        lax.fori_loop(0, _CHUNK, _zero_row, 0)
        base_row = sid * _ROWS_BASE
        for cpy in range(_ROWS_BASE // _CHUNK):
            pltpu.sync_copy(rows_v, h_sh.at[pl.ds(base_row + cpy * _CHUNK, _CHUNK)])
        rem = _ROWS_BASE - (_ROWS_BASE // _CHUNK) * _CHUNK

        @pl.when(sid < _NS - 1)
        def _zero_tail_base():
            pltpu.sync_copy(rows_v.at[pl.ds(0, rem)],
                            h_sh.at[pl.ds(base_row + _ROWS_BASE - rem, rem)])

        @pl.when(sid == _NS - 1)
        def _zero_tail_last():
            pltpu.sync_copy(rows_v, h_sh.at[pl.ds(base_row + _ROWS_BASE - rem, _CHUNK)])
        plsc.subcore_barrier()

        def body(c, carry):
            base = wid * ept + c * _CHUNK
            pltpu.sync_copy(col_hbm.at[pl.ds(base, _CHUNK)], cidx_v)
            pltpu.sync_copy(row_hbm.at[pl.ds(base, _CHUNK)], ridx_v)
            pltpu.sync_copy(vals_hbm.at[pl.ds(base, _CHUNK)], vals_v)
            pltpu.async_copy(x_hbm.at[cidx_v], rows_v, sem).wait()

            def scale(g, inner):
                val16 = vals_v[pl.ds(g * _LANES, _LANES)]
                for e in range(_LANES):
                    bc = val16[e]
                    r = g * _LANES + e
                    for j in range(_DIM // _LANES):
                        sl = pl.ds(j * _LANES, _LANES)
                        rows_v[r, sl] = rows_v[r, sl] * bc
                return inner
            lax.fori_loop(0, _CHUNK // _LANES, scale, 0)

            pltpu.sync_copy(rows_v, h_sh.at[ridx_v], add=True)
            return carry
        lax.fori_loop(0, n_chunks, body, 0)

        plsc.subcore_barrier()

        @pl.when(sid < _NS - 1)
        def _write_base():
            pltpu.sync_copy(h_sh.at[pl.ds(base_row, _ROWS_BASE)],
                            out_hbm.at[cid, pl.ds(base_row, _ROWS_BASE)])

        @pl.when(sid == _NS - 1)
        def _write_last():
            pltpu.sync_copy(h_sh.at[pl.ds(base_row, _ROWS_LAST)],
                            out_hbm.at[cid, pl.ds(base_row, _ROWS_LAST)])

    return agg(x, row, col, vals)


_BN = 1000


def _gate_body(hp_ref, x_ref, w_ref, b_ref, o_ref):
    h = hp_ref[0] + hp_ref[1]
    z = jnp.sum(h * w_ref[...], axis=1, keepdims=True) + b_ref[0, 0]
    g = jax.nn.sigmoid(z)
    o_ref[...] = g * h + (1.0 - g) * x_ref[...]


def _gate(hp, x, W_gate, b_gate):
    wt = W_gate.reshape(1, _DIM)
    bb = b_gate.reshape(1, 1)
    grid = _N // _BN
    return pl.pallas_call(
        _gate_body,
        grid=(grid,),
        in_specs=[
            pl.BlockSpec((_NC, _BN, _DIM), lambda i: (0, i, 0)),
            pl.BlockSpec((_BN, _DIM), lambda i: (i, 0)),
            pl.BlockSpec((1, _DIM), lambda i: (0, 0)),
            pl.BlockSpec(memory_space=pltpu.SMEM),
        ],
        out_specs=pl.BlockSpec((_BN, _DIM), lambda i: (i, 0)),
        out_shape=jax.ShapeDtypeStruct((_N, _DIM), jnp.float32),
    )(hp, x, wt, bb)


def kernel(x, adj_indices, adj_values, W_gate, b_gate):
    row = adj_indices[0].astype(jnp.int32)
    col = adj_indices[1].astype(jnp.int32)
    vals = adj_values.astype(jnp.float32)
    e = row.shape[0]
    unit = _NW * _CHUNK
    e_pad = ((e + unit - 1) // unit) * unit
    pad = e_pad - e
    if pad:
        spread = jnp.arange(pad, dtype=jnp.int32) % _N
        row = jnp.concatenate([row, spread])
        col = jnp.concatenate([col, spread])
        vals = jnp.concatenate([vals, jnp.zeros((pad,), jnp.float32)])
    hp = _sc_aggregate(x, row, col, vals)
    return _gate(hp, x, W_gate, b_gate)
